# baseline (speedup 1.0000x reference)
import functools

import jax
import jax.numpy as jnp
from jax import lax
from jax.experimental import pallas as pl
from jax.experimental.pallas import tpu as pltpu
from jax.experimental.pallas import tpu_sc as plsc

NC = 2
NS = 16
NW = NC * NS
CHUNK = 128


def _emb_body(table_hbm, idx_hbm, out_hbm, idx_v, rows_v, sem):
    wid = lax.axis_index("s") * NC + lax.axis_index("c")
    n_chunks = idx_hbm.shape[1]
    pltpu.sync_copy(idx_hbm.at[wid], idx_v)

    def body(j, carry):
        pltpu.async_copy(table_hbm.at[idx_v.at[j]], rows_v, sem).wait()
        base = (wid * n_chunks + j) * CHUNK
        pltpu.sync_copy(rows_v, out_hbm.at[pl.ds(base, CHUNK)])
        return carry

    lax.fori_loop(0, n_chunks, body, 0)


def kernel(input, weight):
    B, F = input.shape
    D = weight.shape[1]
    total = B * F
    n_chunks = total // (NW * CHUNK)
    idx = input.reshape(NW, n_chunks, CHUNK)

    mesh = plsc.VectorSubcoreMesh(core_axis_name="c", subcore_axis_name="s")
    k = functools.partial(
        pl.kernel,
        mesh=mesh,
        compiler_params=pltpu.CompilerParams(use_tc_tiling_on_sc=False),
        out_type=jax.ShapeDtypeStruct((total, D), weight.dtype),
        scratch_types=[
            pltpu.VMEM((n_chunks, CHUNK), jnp.int32),
            pltpu.VMEM((CHUNK, D), jnp.float32),
            pltpu.SemaphoreType.DMA,
        ],
    )(_emb_body)
    out = k(weight, idx)
    return out.reshape(B, F, D)


# speedup vs baseline: 1.0785x; 1.0785x over previous
import functools

import jax
import jax.numpy as jnp
from jax import lax
from jax.experimental import pallas as pl
from jax.experimental.pallas import tpu as pltpu
from jax.experimental.pallas import tpu_sc as plsc

NC = 2
NS = 16
NW = NC * NS
CHUNK = 128
NBUF = 8
K = 4


def _emb_body(table_hbm, idx_hbm, out_hbm, idx_v, rows_v, gsem, wsem):
    wid = lax.axis_index("s") * NC + lax.axis_index("c")
    n_chunks = idx_hbm.shape[1]
    n_outer = n_chunks // NBUF
    pltpu.sync_copy(idx_hbm.at[wid], idx_v)

    def step(j, b, first_outer, last_outer):
        pltpu.make_async_copy(
            table_hbm.at[idx_v.at[j]], rows_v.at[b], gsem.at[b]).wait()
        base = (wid * n_chunks + j) * CHUNK
        pltpu.async_copy(rows_v.at[b], out_hbm.at[pl.ds(base, CHUNK)],
                         wsem.at[b])
        if not (last_outer and b >= NBUF - K):
            b2 = (b + K) % NBUF
            if not (first_outer and b < NBUF - K):
                pltpu.make_async_copy(
                    rows_v.at[b2], out_hbm.at[pl.ds(0, CHUNK)],
                    wsem.at[b2]).wait()
            pltpu.async_copy(table_hbm.at[idx_v.at[j + K]], rows_v.at[b2],
                             gsem.at[b2])

    for b in range(K):
        pltpu.async_copy(table_hbm.at[idx_v.at[b]], rows_v.at[b], gsem.at[b])

    for b in range(NBUF):
        step(b, b, True, False)

    def outer(g, carry):
        for b in range(NBUF):
            step(g * NBUF + b, b, False, False)
        return carry

    lax.fori_loop(1, n_outer - 1, outer, 0)

    for b in range(NBUF):
        step((n_outer - 1) * NBUF + b, b, False, True)

    for b in range(NBUF):
        pltpu.make_async_copy(
            rows_v.at[b], out_hbm.at[pl.ds(0, CHUNK)], wsem.at[b]).wait()


def kernel(input, weight):
    B, F = input.shape
    D = weight.shape[1]
    total = B * F
    n_chunks = total // (NW * CHUNK)
    idx = input.reshape(NW, n_chunks, CHUNK)

    mesh = plsc.VectorSubcoreMesh(core_axis_name="c", subcore_axis_name="s")
    k = functools.partial(
        pl.kernel,
        mesh=mesh,
        compiler_params=pltpu.CompilerParams(use_tc_tiling_on_sc=False),
        out_type=jax.ShapeDtypeStruct((total, D), weight.dtype),
        scratch_types=[
            pltpu.VMEM((n_chunks, CHUNK), jnp.int32),
            pltpu.VMEM((NBUF, CHUNK, D), jnp.float32),
            pltpu.SemaphoreType.DMA((NBUF,)),
            pltpu.SemaphoreType.DMA((NBUF,)),
        ],
    )(_emb_body)
    out = k(weight, idx)
    return out.reshape(B, F, D)


# speedup vs baseline: 1.0794x; 1.0008x over previous
import functools

import jax
import jax.numpy as jnp
from jax import lax
from jax.experimental import pallas as pl
from jax.experimental.pallas import tpu as pltpu
from jax.experimental.pallas import tpu_sc as plsc

NC = 2
NS = 16
NW = NC * NS
CHUNK = 128
NBUF = 8
K = 4


def _emb_body(table_hbm, idx_hbm, out_hbm, idx_v, rows_v, gsem, wsem):
    wid = lax.axis_index("s") * NC + lax.axis_index("c")
    n_chunks = idx_v.shape[0] // CHUNK
    n_outer = n_chunks // NBUF
    pltpu.sync_copy(idx_hbm.at[pl.ds(wid * idx_v.shape[0], idx_v.shape[0])],
                    idx_v)

    def step(j, b, first_outer, last_outer):
        pltpu.make_async_copy(
            table_hbm.at[idx_v.at[pl.ds(j * CHUNK, CHUNK)]], rows_v.at[b],
            gsem.at[b]).wait()
        base = (wid * n_chunks + j) * CHUNK
        pltpu.async_copy(rows_v.at[b], out_hbm.at[pl.ds(base, CHUNK)],
                         wsem.at[b])
        if not (last_outer and b >= NBUF - K):
            b2 = (b + K) % NBUF
            if not (first_outer and b < NBUF - K):
                pltpu.make_async_copy(
                    rows_v.at[b2], out_hbm.at[pl.ds(0, CHUNK)],
                    wsem.at[b2]).wait()
            pltpu.async_copy(
                table_hbm.at[idx_v.at[pl.ds((j + K) * CHUNK, CHUNK)]],
                rows_v.at[b2], gsem.at[b2])

    for b in range(K):
        pltpu.async_copy(table_hbm.at[idx_v.at[pl.ds(b * CHUNK, CHUNK)]],
                         rows_v.at[b], gsem.at[b])

    for b in range(NBUF):
        step(b, b, True, False)

    def outer(g, carry):
        for b in range(NBUF):
            step(g * NBUF + b, b, False, False)
        return carry

    lax.fori_loop(1, n_outer - 1, outer, 0)

    for b in range(NBUF):
        step((n_outer - 1) * NBUF + b, b, False, True)

    for b in range(NBUF):
        pltpu.make_async_copy(
            rows_v.at[b], out_hbm.at[pl.ds(0, CHUNK)], wsem.at[b]).wait()


def kernel(input, weight):
    B, F = input.shape
    D = weight.shape[1]
    total = B * F
    n_chunks = total // (NW * CHUNK)
    idx = input.reshape(total)

    mesh = plsc.VectorSubcoreMesh(core_axis_name="c", subcore_axis_name="s")
    k = functools.partial(
        pl.kernel,
        mesh=mesh,
        compiler_params=pltpu.CompilerParams(use_tc_tiling_on_sc=False),
        out_type=jax.ShapeDtypeStruct((total, D), weight.dtype),
        scratch_types=[
            pltpu.VMEM((n_chunks * CHUNK,), jnp.int32),
            pltpu.VMEM((NBUF, CHUNK, D), jnp.float32),
            pltpu.SemaphoreType.DMA((NBUF,)),
            pltpu.SemaphoreType.DMA((NBUF,)),
        ],
    )(_emb_body)
    out = k(weight, idx)
    return out.reshape(B, F, D)
